# baseline (speedup 1.0000x reference)
import jax
import jax.numpy as jnp
from jax.experimental import pallas as pl
from jax.experimental.pallas import tpu as pltpu

VOCAB = 1000
BATCH = 1024
HIST = 50
BLOCK_B = 128


def _onehot_block(ids_ref, out_ref):
    ids = ids_ref[:, :]
    iota = jax.lax.broadcasted_iota(jnp.int32, (BLOCK_B, HIST, VOCAB), 2)
    out_ref[:, :, :] = (iota == ids[:, :, None]).astype(jnp.float32)


def kernel(input):
    ids = input.astype(jnp.int32)
    return pl.pallas_call(
        _onehot_block,
        grid=(BATCH // BLOCK_B,),
        in_specs=[pl.BlockSpec((BLOCK_B, HIST), lambda i: (i, 0))],
        out_specs=pl.BlockSpec((BLOCK_B, HIST, VOCAB), lambda i: (i, 0, 0)),
        out_shape=jax.ShapeDtypeStruct((BATCH, HIST, VOCAB), jnp.float32),
        compiler_params=pltpu.CompilerParams(
            dimension_semantics=("parallel",)),
    )(ids)


# speedup vs baseline: 1.0042x; 1.0042x over previous
import jax
import jax.numpy as jnp
from jax.experimental import pallas as pl
from jax.experimental.pallas import tpu as pltpu

VOCAB = 1000
BATCH = 1024
HIST = 50
BLOCK_B = 16
N_BUF = 8
GRID = BATCH // BLOCK_B


def _onehot_body(ids_ref, out_ref, scratch, sems):
    i = pl.program_id(0)
    slot = jax.lax.rem(i, N_BUF)
    ids = ids_ref[pl.ds(i * BLOCK_B, BLOCK_B), :]
    iota = jax.lax.broadcasted_iota(jnp.int32, (BLOCK_B, HIST, VOCAB), 2)

    for k in range(N_BUF):
        @pl.when(slot == k)
        def _(k=k):
            @pl.when(i >= N_BUF)
            def _():
                pltpu.make_async_copy(
                    scratch.at[k],
                    out_ref.at[pl.ds((i - N_BUF) * BLOCK_B, BLOCK_B)],
                    sems.at[k],
                ).wait()

            scratch[k] = (iota == ids[:, :, None]).astype(jnp.float32)
            pltpu.make_async_copy(
                scratch.at[k],
                out_ref.at[pl.ds(i * BLOCK_B, BLOCK_B)],
                sems.at[k],
            ).start()

    @pl.when(i == GRID - 1)
    def _():
        for k in range(N_BUF):
            pltpu.make_async_copy(
                scratch.at[k],
                out_ref.at[pl.ds(0, BLOCK_B)],
                sems.at[k],
            ).wait()


def kernel(input):
    ids = input.astype(jnp.int32)
    return pl.pallas_call(
        _onehot_body,
        grid=(GRID,),
        in_specs=[pl.BlockSpec(memory_space=pltpu.MemorySpace.VMEM)],
        out_specs=pl.BlockSpec(memory_space=pltpu.MemorySpace.HBM),
        out_shape=jax.ShapeDtypeStruct((BATCH, HIST, VOCAB), jnp.float32),
        scratch_shapes=[
            pltpu.VMEM((N_BUF, BLOCK_B, HIST, VOCAB), jnp.float32),
            pltpu.SemaphoreType.DMA((N_BUF,)),
        ],
    )(ids)
